# baseline (speedup 1.0000x reference)
import functools

import jax
import jax.numpy as jnp
from jax import lax
from jax.experimental import pallas as pl
from jax.experimental.pallas import tpu as pltpu
from jax.experimental.pallas import tpu_sc as plsc


def kernel(input_ids, table):
    batch_size, seq_len = input_ids.shape
    max_len, d_model = table.shape

    info = plsc.get_sparse_core_info()
    nc, ns = info.num_cores, info.num_subcores
    nw = nc * ns
    rows_per_w = seq_len // nw
    chunk = 32
    n_chunks = rows_per_w // chunk

    mesh = plsc.VectorSubcoreMesh(core_axis_name="c", subcore_axis_name="s")

    @functools.partial(
        pl.kernel,
        mesh=mesh,
        out_type=jax.ShapeDtypeStruct((batch_size, seq_len, d_model), table.dtype),
        scratch_types=[
            pltpu.VMEM((chunk, d_model), jnp.float32),
        ],
    )
    def sc_copy(table_hbm, out_hbm, buf):
        wid = lax.axis_index("s") * nc + lax.axis_index("c")
        base = wid * rows_per_w
        for i in range(n_chunks):
            start = base + i * chunk
            pltpu.sync_copy(table_hbm.at[pl.ds(start, chunk)], buf)
            for b in range(batch_size):
                pltpu.sync_copy(buf, out_hbm.at[b, pl.ds(start, chunk)])

    return sc_copy(table)


# speedup vs baseline: 1.0495x; 1.0495x over previous
import functools

import jax
import jax.numpy as jnp
from jax import lax
from jax.experimental import pallas as pl
from jax.experimental.pallas import tpu as pltpu
from jax.experimental.pallas import tpu_sc as plsc


def kernel(input_ids, table):
    batch_size, seq_len = input_ids.shape
    max_len, d_model = table.shape

    info = plsc.get_sparse_core_info()
    nc, ns = info.num_cores, info.num_subcores
    nw = nc * ns
    rows_per_w = seq_len // nw
    chunk = 32
    n_chunks = rows_per_w // chunk

    mesh = plsc.VectorSubcoreMesh(core_axis_name="c", subcore_axis_name="s")

    @functools.partial(
        pl.kernel,
        mesh=mesh,
        out_type=jax.ShapeDtypeStruct((batch_size, seq_len, d_model), table.dtype),
        scratch_types=[
            pltpu.VMEM((2, chunk, d_model), jnp.float32),
            pltpu.SemaphoreType.DMA,
            pltpu.SemaphoreType.DMA,
        ],
    )
    def sc_copy(table_hbm, out_hbm, bufs, insem, outsem):
        wid = lax.axis_index("s") * nc + lax.axis_index("c")
        base = wid * rows_per_w

        def cp_in(i):
            start = base + i * chunk
            return pltpu.async_copy(
                table_hbm.at[pl.ds(start, chunk)], bufs.at[i % 2], insem
            )

        def cp_out(i, b):
            start = base + i * chunk
            return pltpu.async_copy(
                bufs.at[i % 2], out_hbm.at[b, pl.ds(start, chunk)], outsem
            )

        h_in = [None] * n_chunks
        h_out = [None] * n_chunks
        h_in[0] = cp_in(0)
        for i in range(n_chunks):
            if i + 1 < n_chunks:
                if i >= 1:
                    for h in h_out[i - 1]:
                        h.wait()
                h_in[i + 1] = cp_in(i + 1)
            h_in[i].wait()
            h_out[i] = [cp_out(i, b) for b in range(batch_size)]
        for i in (n_chunks - 2, n_chunks - 1):
            for h in h_out[i]:
                h.wait()

    return sc_copy(table)
